# baseline (speedup 1.0000x reference)
import functools

import jax
import jax.numpy as jnp
from jax.experimental import pallas as pl
from jax.experimental.pallas import tpu as pltpu

TOKENS = 32768
HIDDEN = 768
E = 64
TOPK = 8

BT = 512


def _router_block(x_ref, w_ref, b_ref, logits_ref, wts_ref, idx_ref, mask_ref):
    x = x_ref[...]
    w = w_ref[...]
    logits = jnp.dot(x, w, preferred_element_type=jnp.float32) + b_ref[...]
    logits_ref[...] = logits

    m = jnp.max(logits, axis=-1, keepdims=True)
    e = jnp.exp(logits - m)
    p = e / jnp.sum(e, axis=-1, keepdims=True)

    col = jax.lax.broadcasted_iota(jnp.int32, (BT, E), 1)
    vals = []
    idxs = []
    for _ in range(TOPK):
        v = jnp.max(p, axis=-1, keepdims=True)
        is_max = p >= v
        i = jnp.min(jnp.where(is_max, col, E), axis=-1, keepdims=True)
        vals.append(v)
        idxs.append(i)
        p = jnp.where(col == i, -1.0, p)
    vals = jnp.concatenate(vals, axis=-1)
    idxs = jnp.concatenate(idxs, axis=-1)

    wts_ref[...] = vals / jnp.sum(vals, axis=-1, keepdims=True)
    idx_ref[...] = idxs

    eid = jax.lax.broadcasted_iota(jnp.int32, (BT, TOPK, E), 2)
    mask_ref[...] = (idxs[:, :, None] == eid).astype(jnp.int32)


@jax.jit
def kernel(hidden_states, W, b):
    grid = (TOKENS // BT,)
    out_shapes = (
        jax.ShapeDtypeStruct((TOKENS, E), jnp.float32),
        jax.ShapeDtypeStruct((TOKENS, TOPK), jnp.float32),
        jax.ShapeDtypeStruct((TOKENS, TOPK), jnp.int32),
        jax.ShapeDtypeStruct((TOKENS, TOPK, E), jnp.int32),
    )
    b2 = b.reshape(1, E)
    return pl.pallas_call(
        _router_block,
        grid=grid,
        in_specs=[
            pl.BlockSpec((BT, HIDDEN), lambda i: (i, 0)),
            pl.BlockSpec((HIDDEN, E), lambda i: (0, 0)),
            pl.BlockSpec((1, E), lambda i: (0, 0)),
        ],
        out_specs=(
            pl.BlockSpec((BT, E), lambda i: (i, 0)),
            pl.BlockSpec((BT, TOPK), lambda i: (i, 0)),
            pl.BlockSpec((BT, TOPK), lambda i: (i, 0)),
            pl.BlockSpec((BT, TOPK, E), lambda i: (i, 0, 0)),
        ),
        out_shape=out_shapes,
    )(hidden_states, W, b2)


# speedup vs baseline: 1.2943x; 1.2943x over previous
import jax
import jax.numpy as jnp
from jax.experimental import pallas as pl

TOKENS = 32768
HIDDEN = 768
E = 64
TOPK = 8

BT = 512


def _router_block(x_ref, w_ref, b_ref, logits_ref, wts_ref, idx_ref, mask_ref):
    logits = jnp.dot(x_ref[...], w_ref[...], preferred_element_type=jnp.float32)
    logits = logits + b_ref[...]
    logits_ref[...] = logits

    lane_f = jax.lax.broadcasted_iota(jnp.int32, (BT, E), 1).astype(jnp.float32)
    keys = logits

    ms = []
    ids = []
    for k in range(TOPK):
        m = jnp.max(keys, axis=-1, keepdims=True)
        sel = keys == m
        mask_ref[:, k, :] = sel.astype(jnp.int32)
        ms.append(m)
        ids.append(jnp.sum(jnp.where(sel, lane_f, 0.0), axis=-1, keepdims=True))
        if k + 1 < TOPK:
            keys = jnp.where(sel, -jnp.inf, keys)

    mtop = jnp.concatenate(ms, axis=-1)
    idx_ref[...] = jnp.concatenate(ids, axis=-1).astype(jnp.int32)

    ex = jnp.exp(mtop - ms[0])
    wts_ref[...] = ex / jnp.sum(ex, axis=-1, keepdims=True)


@jax.jit
def kernel(hidden_states, W, b):
    grid = (TOKENS // BT,)
    out_shapes = (
        jax.ShapeDtypeStruct((TOKENS, E), jnp.float32),
        jax.ShapeDtypeStruct((TOKENS, TOPK), jnp.float32),
        jax.ShapeDtypeStruct((TOKENS, TOPK), jnp.int32),
        jax.ShapeDtypeStruct((TOKENS, TOPK, E), jnp.int32),
    )
    b2 = b.reshape(1, E)
    return pl.pallas_call(
        _router_block,
        grid=grid,
        in_specs=[
            pl.BlockSpec((BT, HIDDEN), lambda i: (i, 0)),
            pl.BlockSpec((HIDDEN, E), lambda i: (0, 0)),
            pl.BlockSpec((1, E), lambda i: (0, 0)),
        ],
        out_specs=(
            pl.BlockSpec((BT, E), lambda i: (i, 0)),
            pl.BlockSpec((BT, TOPK), lambda i: (i, 0)),
            pl.BlockSpec((BT, TOPK), lambda i: (i, 0)),
            pl.BlockSpec((BT, TOPK, E), lambda i: (i, 0, 0)),
        ),
        out_shape=out_shapes,
    )(hidden_states, W, b2)


# speedup vs baseline: 1.6587x; 1.2815x over previous
import jax
import jax.numpy as jnp
from jax.experimental import pallas as pl

TOKENS = 32768
HIDDEN = 768
E = 64
TOPK = 8

BT = 512


def _router_block(x_ref, w_ref, b_ref, logits_ref, wts_ref, idx_ref, mask_ref):
    logits = jnp.dot(x_ref[...], w_ref[...], preferred_element_type=jnp.float32)
    logits = logits + b_ref[...]
    logits_ref[...] = logits

    lane_f = jax.lax.broadcasted_iota(jnp.int32, (BT, E), 1).astype(jnp.float32)
    keys = logits

    ms = []
    ids = []
    sels = []
    for k in range(TOPK):
        m = jnp.max(keys, axis=-1, keepdims=True)
        sel = keys == m
        sels.append(sel.astype(jnp.int32))
        ms.append(m)
        ids.append(jnp.sum(jnp.where(sel, lane_f, 0.0), axis=-1, keepdims=True))
        if k + 1 < TOPK:
            keys = jnp.where(sel, -jnp.inf, keys)
    mask_ref[...] = jnp.concatenate(sels, axis=-1)

    mtop = jnp.concatenate(ms, axis=-1)
    idx_ref[...] = jnp.concatenate(ids, axis=-1).astype(jnp.int32)

    ex = jnp.exp(mtop - ms[0])
    wts_ref[...] = ex / jnp.sum(ex, axis=-1, keepdims=True)


@jax.jit
def kernel(hidden_states, W, b):
    grid = (TOKENS // BT,)
    out_shapes = (
        jax.ShapeDtypeStruct((TOKENS, E), jnp.float32),
        jax.ShapeDtypeStruct((TOKENS, TOPK), jnp.float32),
        jax.ShapeDtypeStruct((TOKENS, TOPK), jnp.int32),
        jax.ShapeDtypeStruct((TOKENS, TOPK * E), jnp.int32),
    )
    b2 = b.reshape(1, E)
    logits, wts, idx, mask2 = pl.pallas_call(
        _router_block,
        grid=grid,
        in_specs=[
            pl.BlockSpec((BT, HIDDEN), lambda i: (i, 0)),
            pl.BlockSpec((HIDDEN, E), lambda i: (0, 0)),
            pl.BlockSpec((1, E), lambda i: (0, 0)),
        ],
        out_specs=(
            pl.BlockSpec((BT, E), lambda i: (i, 0)),
            pl.BlockSpec((BT, TOPK), lambda i: (i, 0)),
            pl.BlockSpec((BT, TOPK), lambda i: (i, 0)),
            pl.BlockSpec((BT, TOPK * E), lambda i: (i, 0)),
        ),
        out_shape=out_shapes,
    )(hidden_states, W, b2)
    return logits, wts, idx, mask2.reshape(TOKENS, TOPK, E)


# speedup vs baseline: 4.0389x; 2.4351x over previous
import jax
import jax.numpy as jnp
from jax.experimental import pallas as pl

TOKENS = 32768
HIDDEN = 768
E = 64
TOPK = 8

BT = 512


def _router_block(x_ref, w_ref, b_ref, logits_ref, wts_ref, idx_ref, mask_ref):
    logits = jnp.dot(x_ref[...], w_ref[...], preferred_element_type=jnp.float32)
    logits = logits + b_ref[...]
    lt = logits.T
    logits_ref[...] = lt

    sub_f = jax.lax.broadcasted_iota(jnp.int32, (E, BT), 0).astype(jnp.float32)
    keys = lt

    ms = []
    ids = []
    for k in range(TOPK):
        m = jnp.max(keys, axis=0, keepdims=True)
        sel = keys == m
        mask_ref[k, :, :] = sel.astype(jnp.int32)
        ms.append(m)
        ids.append(jnp.sum(jnp.where(sel, sub_f, 0.0), axis=0, keepdims=True))
        if k + 1 < TOPK:
            keys = jnp.where(sel, -jnp.inf, keys)

    mtop = jnp.concatenate(ms, axis=0)
    idx_ref[...] = jnp.concatenate(ids, axis=0).astype(jnp.int32)

    ex = jnp.exp(mtop - ms[0])
    wts_ref[...] = ex / jnp.sum(ex, axis=0, keepdims=True)


@jax.jit
def kernel(hidden_states, W, b):
    grid = (TOKENS // BT,)
    out_shapes = (
        jax.ShapeDtypeStruct((E, TOKENS), jnp.float32),
        jax.ShapeDtypeStruct((TOPK, TOKENS), jnp.float32),
        jax.ShapeDtypeStruct((TOPK, TOKENS), jnp.int32),
        jax.ShapeDtypeStruct((TOPK, E, TOKENS), jnp.int32),
    )
    b2 = b.reshape(1, E)
    logits_t, wts_t, idx_t, mask_t = pl.pallas_call(
        _router_block,
        grid=grid,
        in_specs=[
            pl.BlockSpec((BT, HIDDEN), lambda i: (i, 0)),
            pl.BlockSpec((HIDDEN, E), lambda i: (0, 0)),
            pl.BlockSpec((1, E), lambda i: (0, 0)),
        ],
        out_specs=(
            pl.BlockSpec((E, BT), lambda i: (0, i)),
            pl.BlockSpec((TOPK, BT), lambda i: (0, i)),
            pl.BlockSpec((TOPK, BT), lambda i: (0, i)),
            pl.BlockSpec((TOPK, E, BT), lambda i: (0, 0, i)),
        ),
        out_shape=out_shapes,
    )(hidden_states, W, b2)
    return (
        logits_t.T,
        wts_t.T,
        idx_t.T,
        jnp.transpose(mask_t, (2, 0, 1)),
    )
